# baseline (speedup 1.0000x reference)
import jax
import jax.numpy as jnp
from jax import lax
from jax.experimental import pallas as pl
from jax.experimental.pallas import tpu as pltpu

V, H, E = 32, 1024, 1536
N = 64
RB = 8
LA = 4


def _body(ng_s, uniq_s, start_s, count_s, order_s, table_any, out_any,
          vbuf, isem, osem):
    ng = ng_s[0]

    def fetch(g, slot):
        pltpu.make_async_copy(
            table_any.at[uniq_s[g]], vbuf.at[slot], isem.at[slot]).start()

    def drain_writes(g, slot):
        def dbody(k, c):
            pltpu.make_async_copy(
                vbuf.at[slot], out_any.at[0], osem.at[slot]).wait()
            return c
        lax.fori_loop(0, count_s[g], dbody, 0)

    def prologue(g, c):
        @pl.when(g < ng)
        def _():
            fetch(g, g)
        return c

    lax.fori_loop(0, LA, prologue, 0)

    def gloop(g, c):
        slot = lax.rem(g, RB)

        gd = g + LA - RB

        @pl.when(gd >= 0)
        def _():
            drain_writes(gd, lax.rem(gd, RB))

        @pl.when(g + LA < ng)
        def _():
            fetch(g + LA, lax.rem(g + LA, RB))

        pltpu.make_async_copy(
            table_any.at[uniq_s[g]], vbuf.at[slot], isem.at[slot]).wait()

        def wbody(k, c2):
            pltpu.make_async_copy(
                vbuf.at[slot], out_any.at[order_s[start_s[g] + k]],
                osem.at[slot]).start()
            return c2

        lax.fori_loop(0, count_s[g], wbody, 0)
        return c

    lax.fori_loop(0, ng, gloop, 0)

    def fdrain(g, c):
        @pl.when(g + RB - LA >= ng)
        def _():
            drain_writes(g, lax.rem(g, RB))
        return c

    lax.fori_loop(0, ng, fdrain, 0)


def _tc_gather(ng, uniq, start, count, order, table):
    return pl.pallas_call(
        _body,
        grid_spec=pltpu.PrefetchScalarGridSpec(
            num_scalar_prefetch=5,
            grid=(1,),
            in_specs=[pl.BlockSpec(memory_space=pl.ANY)],
            out_specs=pl.BlockSpec(memory_space=pl.ANY),
            scratch_shapes=[
                pltpu.VMEM((RB, H, E), jnp.float32),
                pltpu.SemaphoreType.DMA((RB,)),
                pltpu.SemaphoreType.DMA((RB,)),
            ],
        ),
        out_shape=jax.ShapeDtypeStruct((N, H, E), jnp.float32),
    )(ng, uniq, start, count, order, table)


def kernel(cat_ids, W):
    ids = cat_ids.astype(jnp.int32)
    order = jnp.argsort(ids).astype(jnp.int32)
    sids = ids[order]
    iarange = jnp.arange(N, dtype=jnp.int32)
    is_new = jnp.concatenate(
        [jnp.ones((1,), jnp.bool_), sids[1:] != sids[:-1]])
    ng = jnp.sum(is_new, dtype=jnp.int32)[None]
    start = jnp.nonzero(is_new, size=N, fill_value=N)[0].astype(jnp.int32)
    count = jnp.append(start[1:], jnp.int32(N)) - start
    uniq = sids[jnp.clip(start, 0, N - 1)]
    return _tc_gather(ng, uniq, start, count, order, W)


# speedup vs baseline: 1.0039x; 1.0039x over previous
import jax
import jax.numpy as jnp
from jax.experimental import pallas as pl
from jax.experimental.pallas import tpu as pltpu

V, H, E = 32, 1024, 1536
N = 64


def _copy_body(sids_smem, order_smem, in_ref, out_ref):
    out_ref[...] = in_ref[...]


def _tc_gather(sids, order, table):
    return pl.pallas_call(
        _copy_body,
        grid_spec=pltpu.PrefetchScalarGridSpec(
            num_scalar_prefetch=2,
            grid=(N,),
            in_specs=[
                pl.BlockSpec((1, H, E), lambda i, sids, order: (sids[i], 0, 0)),
            ],
            out_specs=pl.BlockSpec((1, H, E), lambda i, sids, order: (order[i], 0, 0)),
        ),
        out_shape=jax.ShapeDtypeStruct((N, H, E), jnp.float32),
    )(sids, order, table)


def kernel(cat_ids, W):
    ids = cat_ids.astype(jnp.int32)
    order = jnp.argsort(ids).astype(jnp.int32)
    sids = ids[order]
    return _tc_gather(sids, order, W)
